# Initial kernel scaffold; baseline (speedup 1.0000x reference)
import jax
import jax.numpy as jnp
from jax.experimental import pallas as pl


def kernel(hidden_states, gate_w, w_gate, w_up, w_down):
    raise NotImplementedError("write your pallas kernel here")


# speedup vs baseline: 1.3545x; 1.3545x over previous
import functools

import jax
import jax.numpy as jnp
from jax import lax
from jax.experimental import pallas as pl
from jax.experimental.pallas import tpu as pltpu

T = 2048
D = 768
FFN = 3072
E = 8

_INTERPRET = False


def _logits_body(hs_ref, gw_ref, logits_ref):
    logits_ref[...] = lax.dot_general(
        hs_ref[...], gw_ref[...], (((1,), (1,)), ((), ())),
        preferred_element_type=jnp.float32).astype(jnp.bfloat16)


def _logits(hs, gate_w):
    return pl.pallas_call(
        _logits_body,
        out_shape=jax.ShapeDtypeStruct((T, E), jnp.bfloat16),
        interpret=_INTERPRET,
    )(hs, gate_w)


def _dispatch_body(sel_ref, w_ref, combine_ref):
    lane = lax.broadcasted_iota(jnp.int32, (T, E), 1)

    def _bc(v):
        return jnp.broadcast_to(v, (T, E))

    sel0 = sel_ref[:, 0:1]
    sel1 = sel_ref[:, 1:2]
    w0 = w_ref[:, 0:1].astype(jnp.float32)
    w1 = w_ref[:, 1:2].astype(jnp.float32)
    oh0 = (lane == _bc(sel0)).astype(jnp.float32)
    oh1 = (lane == _bc(sel1)).astype(jnp.float32)
    combine_ref[...] = (oh0 * _bc(w0) + oh1 * _bc(w1)).astype(jnp.bfloat16)


def _dispatch(sel, w):
    return pl.pallas_call(
        _dispatch_body,
        out_shape=jax.ShapeDtypeStruct((T, E), jnp.bfloat16),
        interpret=_INTERPRET,
    )(sel, w)


def _ffn_body(x_ref, wg_ref, wu_ref, wd_ref, comb_ref, out_ref):
    e = pl.program_id(1)
    x = x_ref[...]
    a = lax.dot_general(x, wg_ref[0], (((1,), (1,)), ((), ())),
                        preferred_element_type=jnp.float32).astype(jnp.bfloat16)
    b = lax.dot_general(x, wu_ref[0], (((1,), (1,)), ((), ())),
                        preferred_element_type=jnp.float32).astype(jnp.bfloat16)
    h = (a * jax.nn.sigmoid(a)) * b
    y = lax.dot_general(h, wd_ref[0], (((1,), (1,)), ((), ())),
                        preferred_element_type=jnp.float32).astype(jnp.bfloat16)
    lane = lax.broadcasted_iota(jnp.int32, comb_ref.shape, 1)
    c = jnp.sum(jnp.where(lane == e, comb_ref[...], jnp.bfloat16(0)),
                axis=1, keepdims=True)
    contrib = y * c

    @pl.when(e == 0)
    def _():
        out_ref[...] = contrib

    @pl.when(e > 0)
    def _():
        out_ref[...] = out_ref[...] + contrib


def _dense_moe(hs, w_gate, w_up, w_down, combine):
    tb = 2
    rows = T // tb
    return pl.pallas_call(
        _ffn_body,
        grid=(tb, E),
        in_specs=[
            pl.BlockSpec((rows, D), lambda t, e: (t, 0)),
            pl.BlockSpec((1, FFN, D), lambda t, e: (e, 0, 0)),
            pl.BlockSpec((1, FFN, D), lambda t, e: (e, 0, 0)),
            pl.BlockSpec((1, D, FFN), lambda t, e: (e, 0, 0)),
            pl.BlockSpec((rows, E), lambda t, e: (t, 0)),
        ],
        out_specs=pl.BlockSpec((rows, D), lambda t, e: (t, 0)),
        out_shape=jax.ShapeDtypeStruct((T, D), jnp.bfloat16),
        interpret=_INTERPRET,
    )(hs, w_gate, w_up, w_down, combine)


@functools.partial(jax.jit, static_argnames=())
def kernel(hidden_states, gate_w, w_gate, w_up, w_down):
    bsz, seq, d = hidden_states.shape
    hs = hidden_states.reshape(-1, d)
    logits = (hs @ gate_w.T).astype(jnp.bfloat16)
    p = jax.nn.softmax(logits, axis=1)
    rw_topk, sel = jax.lax.top_k(p, 2)
    rw32 = rw_topk.astype(jnp.float32)
    rw32 = rw32 / rw32.sum(axis=-1, keepdims=True)
    w = rw32.astype(jnp.bfloat16)
    combine = _dispatch(sel, w)
    out = _dense_moe(hs, w_gate, w_up, w_down, combine)
    return out.reshape(bsz, seq, d), logits
